# Initial kernel scaffold; baseline (speedup 1.0000x reference)
import jax
import jax.numpy as jnp
from jax.experimental import pallas as pl


def kernel(x, batch, v):
    raise NotImplementedError("write your pallas kernel here")


# speedup vs baseline: 139.2964x; 139.2964x over previous
import functools

import jax
import jax.numpy as jnp
import numpy as np
from jax import lax
from jax.experimental import pallas as pl
from jax.experimental.pallas import tpu as pltpu
from jax.experimental.pallas import tpu_sc as plsc

NUM_POINTS = 262144
NUM_THETAS = 128
RESOLUTION = 128
RADIUS = 1.1
BATCH_SIZE = 64

NC = 2
NS = 16
LANES = 16
NW = NC * NS
CHUNK = NUM_POINTS // NW
NTV = NUM_THETAS // LANES
ROWS = BATCH_SIZE * RESOLUTION
ROWS_PER_TILE = ROWS // NS

_SCALE = np.float32(RESOLUTION / (2.0 * RADIUS))
_OFFSET = np.float32(RADIUS * RESOLUTION / (2.0 * RADIUS))


def _sc_hist_kernel(x0_hbm, x1_hbm, x2_hbm, b_hbm, v_hbm, out_hbm, x0v, x1v,
                    x2v, bv, hist, wv, idxbuf, accum):
    cid = lax.axis_index("c")
    sid = lax.axis_index("s")
    wid = cid * NS + sid
    base = wid * CHUNK

    zero16 = jnp.zeros((LANES,), jnp.float32)
    iota16 = lax.iota(jnp.int32, LANES)
    ones16 = jnp.ones((LANES,), jnp.float32)

    def _zero_hist():
        def zr(r, carry):
            for jj in range(NTV):
                hist[r, pl.ds(jj * LANES, LANES)] = zero16
            return carry
        lax.fori_loop(0, RESOLUTION, zr, 0)

    _zero_hist()
    for q in range(ROWS_PER_TILE // RESOLUTION):
        pltpu.sync_copy(hist, accum.at[pl.ds((sid * 4 + q) * RESOLUTION, RESOLUTION), :])
    plsc.subcore_barrier()

    for xd_hbm, xdv in ((x0_hbm, x0v), (x1_hbm, x1v), (x2_hbm, x2v)):
        pltpu.sync_copy(xd_hbm.at[pl.ds(base, CHUNK)], xdv)
    pltpu.sync_copy(b_hbm.at[pl.ds(base, CHUNK)], bv)
    pltpu.sync_copy(v_hbm, wv)

    wregs = [wv[i, :] * _SCALE for i in range(3 * NTV)]
    idxt = [iota16 + (jj * LANES) for jj in range(NTV)]

    def _flush(cb):
        rowbase = cb * RESOLUTION
        for jj in range(RESOLUTION // LANES):
            idxbuf[pl.ds(jj * LANES, LANES)] = rowbase + jj * LANES + iota16
        pltpu.sync_copy(hist, accum.at[idxbuf], add=True)
        _zero_hist()

    def gbody(g, cur_b):
        vx0 = x0v[pl.ds(g * LANES, LANES)]
        vx1 = x1v[pl.ds(g * LANES, LANES)]
        vx2 = x2v[pl.ds(g * LANES, LANES)]
        vb = bv[pl.ds(g * LANES, LANES)]
        for k in range(LANES):
            b = vb[k]

            @pl.when(b != cur_b)
            def _(cb=cur_b):
                _flush(cb)

            cur_b = b
            x0 = vx0[k]
            x1 = vx1[k]
            x2 = vx2[k]
            for jj in range(NTV):
                t = (x0 * wregs[jj] + x1 * wregs[NTV + jj]
                     + x2 * wregs[2 * NTV + jj] + _OFFSET)
                ti = t.astype(jnp.int32)
                ti = jnp.minimum(jnp.maximum(ti, 0), RESOLUTION - 1)
                plsc.addupdate_scatter(hist, [ti, idxt[jj]], ones16)
        return cur_b

    first = bv[pl.ds(0, LANES)]
    last_b = lax.fori_loop(0, CHUNK // LANES, gbody, first[0])
    _flush(last_b)

    plsc.subcore_barrier()
    pltpu.sync_copy(
        accum.at[pl.ds(sid * ROWS_PER_TILE, ROWS_PER_TILE), :],
        out_hbm.at[cid, pl.ds(sid * ROWS_PER_TILE, ROWS_PER_TILE), :],
    )


def _make_sc_hist():
    mesh = plsc.VectorSubcoreMesh(core_axis_name="c", subcore_axis_name="s")

    return pl.kernel(
        _sc_hist_kernel,
        mesh=mesh,
        compiler_params=pltpu.CompilerParams(needs_layout_passes=False),
        out_type=jax.ShapeDtypeStruct((NC, ROWS, NUM_THETAS), jnp.float32),
        scratch_types=[
            pltpu.VMEM((CHUNK,), jnp.float32),
            pltpu.VMEM((CHUNK,), jnp.float32),
            pltpu.VMEM((CHUNK,), jnp.float32),
            pltpu.VMEM((CHUNK,), jnp.int32),
            pltpu.VMEM((RESOLUTION, NUM_THETAS), jnp.float32),
            pltpu.VMEM((3 * NTV, LANES), jnp.float32),
            pltpu.VMEM((RESOLUTION,), jnp.int32),
            pltpu.VMEM_SHARED((ROWS, NUM_THETAS), jnp.float32),
        ],
    )


def _tc_cumsum_body(p_ref, o_ref):
    acc = p_ref[0] + p_ref[1]
    for k in range(7):
        step = 1 << k
        shifted = jnp.concatenate(
            [jnp.zeros((step, NUM_THETAS), jnp.float32),
             acc[: RESOLUTION - step, :]], axis=0)
        acc = acc + shifted
    o_ref[0] = acc


_tc_cumsum = pl.pallas_call(
    _tc_cumsum_body,
    grid=(BATCH_SIZE,),
    in_specs=[pl.BlockSpec((NC, RESOLUTION, NUM_THETAS), lambda b: (0, b, 0))],
    out_specs=pl.BlockSpec((1, RESOLUTION, NUM_THETAS), lambda b: (b, 0, 0)),
    out_shape=jax.ShapeDtypeStruct((BATCH_SIZE, RESOLUTION, NUM_THETAS), jnp.float32),
)


def kernel(x, batch, v):
    vr = v.reshape(3 * NTV, LANES)
    xt = x.T
    partials = _make_sc_hist()(xt[0], xt[1], xt[2], batch, vr)
    return _tc_cumsum(partials)
